# baseline (speedup 1.0000x reference)
import jax
import jax.numpy as jnp
from jax import lax
from jax.experimental import pallas as pl
from jax.experimental.pallas import tpu as pltpu
from jax.experimental.pallas import tpu_sc as plsc

N = 10000
E = 320000
D = 128

NC = 2
NS = 16
L = 16
NW = NC * NS
EPW = E // NW
NPAD = 10240
GCHUNK = 64
N0 = 8192
N1 = NPAD - N0
CPW0 = N0 // NS
CPW1 = N1 // NS
NG0 = CPW0 // GCHUNK
NG1 = CPW1 // GCHUNK

BLK = 1000


def _winner_tables_body(ei_hbm, tables_hbm, ei_v, table_v, sem):
    wid = lax.axis_index("s") * NC + lax.axis_index("c")
    base = wid * EPW
    ld = pltpu.async_copy(ei_hbm.at[pl.ds(base, EPW)], ei_v, sem)

    neg1 = jnp.full((L,), -1, jnp.int32)

    def init_body(i, c):
        for u in range(8):
            table_v[pl.ds((i * 8 + u) * L, L)] = neg1
        return c

    lax.fori_loop(0, NPAD // (8 * L), init_body, 0)
    ld.wait()

    lane = lax.iota(jnp.int32, L)
    lane_next = jnp.minimum(lane + 1, L - 1)
    is_top = lane == L - 1
    zero_col = jnp.zeros((L,), jnp.int32)
    one_col = jnp.full((L,), 1, jnp.int32)
    UNROLL = 5

    def edge_body(i, c):
        for u in range(UNROLL):
            v = i * UNROLL + u
            row = v * L + lane
            s = plsc.load_gather(ei_v, [row, zero_col])
            d = plsc.load_gather(ei_v, [row, one_col])
            key = d * L + lane
            key_s, s_s = plsc.sort_key_val(key, s)
            d_s = key_s >> 4
            d_next = d_s.at[lane_next].get(mode="promise_in_bounds")
            is_last = jnp.logical_or(d_s != d_next, is_top)
            plsc.store_scatter(table_v, [d_s], s_s, mask=is_last)
        return c

    lax.fori_loop(0, EPW // (UNROLL * L), edge_body, 0)

    pltpu.sync_copy(table_v, tables_hbm.at[pl.ds(wid * NPAD, NPAD)])


def _merge_gather_body(tables_hbm, xpad_hbm, gathered_hbm,
                       tabs_v, idx_v, rows_v, sem):
    cid = lax.axis_index("c")
    sid = lax.axis_index("s")

    def do(base, cnt, ng):
        loads = [
            pltpu.async_copy(tables_hbm.at[pl.ds(t * NPAD + base, cnt)],
                             tabs_v.at[pl.ds(t * cnt, cnt)], sem)
            for t in range(NW)
        ]
        for ld in loads:
            ld.wait()

        for v in range(cnt // L):
            def merge_t(t, cur):
                for u in range(4):
                    cand = tabs_v[pl.ds((t * 4 + u) * cnt + v * L, L)]
                    cur = jnp.where(cand >= 0, cand, cur)
                return cur

            cur = lax.fori_loop(0, NW // 4, merge_t,
                                jnp.full((L,), -1, jnp.int32))
            clean = jnp.where(cur >= 0, cur, N)
            r, c = (v * L) // GCHUNK, (v * L) % GCHUNK
            idx_v[r, pl.ds(c, L)] = clean

        copies = [
            pltpu.async_copy(xpad_hbm.at[idx_v.at[g]],
                             rows_v.at[pl.ds(g * GCHUNK, GCHUNK)], sem)
            for g in range(ng)
        ]
        for cp in copies:
            cp.wait()
        pltpu.sync_copy(rows_v.at[pl.ds(0, cnt)],
                        gathered_hbm.at[pl.ds(base, cnt)])

    @pl.when(cid == 0)
    def _c0():
        do(sid * CPW0, CPW0, NG0)

    @pl.when(cid == 1)
    def _c1():
        do(N0 + sid * CPW1, CPW1, NG1)


def _mlp_body(g_ref, x_ref, w1_ref, b1_ref, w2_ref, b2_ref, o_ref):
    h = jnp.maximum(
        jnp.dot(g_ref[...], w1_ref[...],
                preferred_element_type=jnp.float32) + b1_ref[...], 0.0)
    y = jnp.maximum(
        jnp.dot(h + x_ref[...], w2_ref[...],
                preferred_element_type=jnp.float32) + b2_ref[...], 0.0)
    o_ref[...] = y


def kernel(node_features, edge_index, W1, b1, W2, b2):
    x_pad = jnp.zeros((NPAD, D), jnp.float32).at[:N, :].set(node_features)

    mesh = plsc.VectorSubcoreMesh(core_axis_name="c", subcore_axis_name="s")
    sc_params = pltpu.CompilerParams(needs_layout_passes=False)
    sc_params_untiled = pltpu.CompilerParams(
        needs_layout_passes=False, use_tc_tiling_on_sc=False)

    tables = pl.kernel(
        _winner_tables_body,
        out_type=jax.ShapeDtypeStruct((NW * NPAD,), jnp.int32),
        mesh=mesh,
        compiler_params=sc_params_untiled,
        scratch_types=[
            pltpu.VMEM((EPW, 2), jnp.int32),
            pltpu.VMEM((NPAD,), jnp.int32),
            pltpu.SemaphoreType.DMA,
        ],
    )(edge_index)

    gathered = pl.kernel(
        _merge_gather_body,
        out_type=jax.ShapeDtypeStruct((NPAD, D), jnp.float32),
        mesh=mesh,
        compiler_params=sc_params,
        scratch_types=[
            pltpu.VMEM((NW * CPW0,), jnp.int32),
            pltpu.VMEM((NG0, GCHUNK), jnp.int32),
            pltpu.VMEM((CPW0, D), jnp.float32),
            pltpu.SemaphoreType.DMA,
        ],
    )(tables, x_pad)

    return pl.pallas_call(
        _mlp_body,
        grid=(N // BLK,),
        in_specs=[
            pl.BlockSpec((BLK, D), lambda i: (i, 0)),
            pl.BlockSpec((BLK, D), lambda i: (i, 0)),
            pl.BlockSpec((D, D), lambda i: (0, 0)),
            pl.BlockSpec((1, D), lambda i: (0, 0)),
            pl.BlockSpec((D, D), lambda i: (0, 0)),
            pl.BlockSpec((1, D), lambda i: (0, 0)),
        ],
        out_specs=pl.BlockSpec((BLK, D), lambda i: (i, 0)),
        out_shape=jax.ShapeDtypeStruct((N, D), jnp.float32),
    )(gathered, node_features, W1, b1.reshape(1, D), W2, b2.reshape(1, D))


# speedup vs baseline: 4.0741x; 4.0741x over previous
import jax
import jax.numpy as jnp
from jax import lax
from jax.experimental import pallas as pl
from jax.experimental.pallas import tpu as pltpu
from jax.experimental.pallas import tpu_sc as plsc

N = 10000
E = 320000
D = 128

NC = 2
NS = 16
L = 16
NW = NC * NS
EPW = E // NW
NPAD = 10240
GCHUNK = 64
N0 = 8192
N1 = NPAD - N0
CPW0 = N0 // NS
CPW1 = N1 // NS
NG0 = CPW0 // GCHUNK
NG1 = CPW1 // GCHUNK

BLK = 1000


def _winner_tables_body(dst_hbm, src_hbm, tables_hbm,
                        dst_v, src_v, table_v, sem):
    wid = lax.axis_index("s") * NC + lax.axis_index("c")
    base = wid * EPW
    ld_d = pltpu.async_copy(dst_hbm.at[pl.ds(base, EPW)], dst_v, sem)
    ld_s = pltpu.async_copy(src_hbm.at[pl.ds(base, EPW)], src_v, sem)

    neg1 = jnp.full((L,), -1, jnp.int32)

    def init_body(i, c):
        for u in range(8):
            table_v[pl.ds((i * 8 + u) * L, L)] = neg1
        return c

    lax.fori_loop(0, NPAD // (8 * L), init_body, 0)
    ld_d.wait()
    ld_s.wait()

    lane = lax.iota(jnp.int32, L)
    lane_next = jnp.minimum(lane + 1, L - 1)
    is_top = lane == L - 1
    UNROLL = 25

    def edge_body(i, c):
        for u in range(UNROLL):
            v = i * UNROLL + u
            d = dst_v[pl.ds(v * L, L)]
            s = src_v[pl.ds(v * L, L)]
            key = d * L + lane
            key_s, s_s = plsc.sort_key_val(key, s)
            d_s = key_s >> 4
            d_next = d_s.at[lane_next].get(mode="promise_in_bounds")
            is_last = jnp.logical_or(d_s != d_next, is_top)
            plsc.store_scatter(table_v, [d_s], s_s, mask=is_last)
        return c

    lax.fori_loop(0, EPW // (UNROLL * L), edge_body, 0)

    pltpu.sync_copy(table_v, tables_hbm.at[pl.ds(wid * NPAD, NPAD)])


def _merge_gather_body(tables_hbm, xpad_hbm, gathered_hbm,
                       tabs_v, idx_v, rows_v, sem):
    cid = lax.axis_index("c")
    sid = lax.axis_index("s")

    def do(base, cnt, ng):
        loads = [
            pltpu.async_copy(tables_hbm.at[pl.ds(t * NPAD + base, cnt)],
                             tabs_v.at[pl.ds(t * cnt, cnt)], sem)
            for t in range(NW)
        ]
        for ld in loads:
            ld.wait()

        for v in range(cnt // L):
            def merge_t(t, cur):
                for u in range(4):
                    cand = tabs_v[pl.ds((t * 4 + u) * cnt + v * L, L)]
                    cur = jnp.where(cand >= 0, cand, cur)
                return cur

            cur = lax.fori_loop(0, NW // 4, merge_t,
                                jnp.full((L,), -1, jnp.int32))
            clean = jnp.where(cur >= 0, cur, N)
            r, c = (v * L) // GCHUNK, (v * L) % GCHUNK
            idx_v[r, pl.ds(c, L)] = clean

        copies = [
            pltpu.async_copy(xpad_hbm.at[idx_v.at[g]],
                             rows_v.at[pl.ds(g * GCHUNK, GCHUNK)], sem)
            for g in range(ng)
        ]
        for cp in copies:
            cp.wait()
        pltpu.sync_copy(rows_v.at[pl.ds(0, cnt)],
                        gathered_hbm.at[pl.ds(base, cnt)])

    @pl.when(cid == 0)
    def _c0():
        do(sid * CPW0, CPW0, NG0)

    @pl.when(cid == 1)
    def _c1():
        do(N0 + sid * CPW1, CPW1, NG1)


def _mlp_body(g_ref, x_ref, w1_ref, b1_ref, w2_ref, b2_ref, o_ref):
    h = jnp.maximum(
        jnp.dot(g_ref[...], w1_ref[...],
                preferred_element_type=jnp.float32) + b1_ref[...], 0.0)
    y = jnp.maximum(
        jnp.dot(h + x_ref[...], w2_ref[...],
                preferred_element_type=jnp.float32) + b2_ref[...], 0.0)
    o_ref[...] = y


def kernel(node_features, edge_index, W1, b1, W2, b2):
    src = edge_index[:, 0]
    dst = edge_index[:, 1]
    x_pad = jnp.zeros((NPAD, D), jnp.float32).at[:N, :].set(node_features)

    mesh = plsc.VectorSubcoreMesh(core_axis_name="c", subcore_axis_name="s")
    sc_params = pltpu.CompilerParams(needs_layout_passes=False)

    tables = pl.kernel(
        _winner_tables_body,
        out_type=jax.ShapeDtypeStruct((NW * NPAD,), jnp.int32),
        mesh=mesh,
        compiler_params=sc_params,
        scratch_types=[
            pltpu.VMEM((EPW,), jnp.int32),
            pltpu.VMEM((EPW,), jnp.int32),
            pltpu.VMEM((NPAD,), jnp.int32),
            pltpu.SemaphoreType.DMA,
        ],
    )(dst, src)

    gathered = pl.kernel(
        _merge_gather_body,
        out_type=jax.ShapeDtypeStruct((NPAD, D), jnp.float32),
        mesh=mesh,
        compiler_params=sc_params,
        scratch_types=[
            pltpu.VMEM((NW * CPW0,), jnp.int32),
            pltpu.VMEM((NG0, GCHUNK), jnp.int32),
            pltpu.VMEM((CPW0, D), jnp.float32),
            pltpu.SemaphoreType.DMA,
        ],
    )(tables, x_pad)

    return pl.pallas_call(
        _mlp_body,
        grid=(N // BLK,),
        in_specs=[
            pl.BlockSpec((BLK, D), lambda i: (i, 0)),
            pl.BlockSpec((BLK, D), lambda i: (i, 0)),
            pl.BlockSpec((D, D), lambda i: (0, 0)),
            pl.BlockSpec((1, D), lambda i: (0, 0)),
            pl.BlockSpec((D, D), lambda i: (0, 0)),
            pl.BlockSpec((1, D), lambda i: (0, 0)),
        ],
        out_specs=pl.BlockSpec((BLK, D), lambda i: (i, 0)),
        out_shape=jax.ShapeDtypeStruct((N, D), jnp.float32),
    )(gathered, node_features, W1, b1.reshape(1, D), W2, b2.reshape(1, D))


# speedup vs baseline: 4.1045x; 1.0075x over previous
import jax
import jax.numpy as jnp
from jax import lax
from jax.experimental import pallas as pl
from jax.experimental.pallas import tpu as pltpu
from jax.experimental.pallas import tpu_sc as plsc

N = 10000
E = 320000
D = 128

NC = 2
NS = 16
L = 16
NW = NC * NS
EPW = E // NW
NPAD = 10240
GCHUNK = 64
N0 = 7168
N1 = NPAD - N0
CPW0 = N0 // NS
CPW1 = N1 // NS
NG0 = CPW0 // GCHUNK
NG1 = CPW1 // GCHUNK

BLK = 1000


def _winner_tables_body(dst_hbm, src_hbm, tables_hbm,
                        dst_v, src_v, table_v, sem):
    wid = lax.axis_index("s") * NC + lax.axis_index("c")
    base = wid * EPW
    ld_d = pltpu.async_copy(dst_hbm.at[pl.ds(base, EPW)], dst_v, sem)
    ld_s = pltpu.async_copy(src_hbm.at[pl.ds(base, EPW)], src_v, sem)

    neg1 = jnp.full((L,), -1, jnp.int32)

    def init_body(i, c):
        for u in range(8):
            table_v[pl.ds((i * 8 + u) * L, L)] = neg1
        return c

    lax.fori_loop(0, NPAD // (8 * L), init_body, 0)
    ld_d.wait()
    ld_s.wait()

    lane = lax.iota(jnp.int32, L)
    lane_next = jnp.minimum(lane + 1, L - 1)
    is_top = lane == L - 1
    UNROLL = 5

    def edge_body(i, c):
        for u in range(UNROLL):
            v = i * UNROLL + u
            d = dst_v[pl.ds(v * L, L)]
            s = src_v[pl.ds(v * L, L)]
            key = d * L + lane
            key_s, s_s = plsc.sort_key_val(key, s)
            d_s = key_s >> 4
            d_next = d_s.at[lane_next].get(mode="promise_in_bounds")
            is_last = jnp.logical_or(d_s != d_next, is_top)
            plsc.store_scatter(table_v, [d_s], s_s, mask=is_last)
        return c

    lax.fori_loop(0, EPW // (UNROLL * L), edge_body, 0)

    pltpu.sync_copy(table_v, tables_hbm.at[pl.ds(wid * NPAD, NPAD)])


def _merge_gather_body(tables_hbm, xpad_hbm, gathered_hbm,
                       tabs_v, idx_v, rows_v, sem):
    cid = lax.axis_index("c")
    sid = lax.axis_index("s")

    def do(base, cnt, ng):
        loads = [
            pltpu.async_copy(tables_hbm.at[pl.ds(t * NPAD + base, cnt)],
                             tabs_v.at[pl.ds(t * cnt, cnt)], sem)
            for t in range(NW)
        ]
        for ld in loads:
            ld.wait()

        for v in range(cnt // L):
            def merge_t(t, cur):
                for u in range(4):
                    cand = tabs_v[pl.ds((t * 4 + u) * cnt + v * L, L)]
                    cur = jnp.where(cand >= 0, cand, cur)
                return cur

            cur = lax.fori_loop(0, NW // 4, merge_t,
                                jnp.full((L,), -1, jnp.int32))
            clean = jnp.where(cur >= 0, cur, N)
            r, c = (v * L) // GCHUNK, (v * L) % GCHUNK
            idx_v[r, pl.ds(c, L)] = clean

        copies = [
            pltpu.async_copy(xpad_hbm.at[idx_v.at[g]],
                             rows_v.at[pl.ds(g * GCHUNK, GCHUNK)], sem)
            for g in range(ng)
        ]
        for cp in copies:
            cp.wait()
        pltpu.sync_copy(rows_v.at[pl.ds(0, cnt)],
                        gathered_hbm.at[pl.ds(base, cnt)])

    @pl.when(cid == 0)
    def _c0():
        do(sid * CPW0, CPW0, NG0)

    @pl.when(cid == 1)
    def _c1():
        do(N0 + sid * CPW1, CPW1, NG1)


def _mlp_body(g_ref, x_ref, w1_ref, b1_ref, w2_ref, b2_ref, o_ref):
    h = jnp.maximum(
        jnp.dot(g_ref[...], w1_ref[...],
                preferred_element_type=jnp.float32) + b1_ref[...], 0.0)
    y = jnp.maximum(
        jnp.dot(h + x_ref[...], w2_ref[...],
                preferred_element_type=jnp.float32) + b2_ref[...], 0.0)
    o_ref[...] = y


def kernel(node_features, edge_index, W1, b1, W2, b2):
    src = edge_index[:, 0]
    dst = edge_index[:, 1]
    x_pad = jnp.zeros((NPAD, D), jnp.float32).at[:N, :].set(node_features)

    mesh = plsc.VectorSubcoreMesh(core_axis_name="c", subcore_axis_name="s")
    sc_params = pltpu.CompilerParams(needs_layout_passes=False)

    tables = pl.kernel(
        _winner_tables_body,
        out_type=jax.ShapeDtypeStruct((NW * NPAD,), jnp.int32),
        mesh=mesh,
        compiler_params=sc_params,
        scratch_types=[
            pltpu.VMEM((EPW,), jnp.int32),
            pltpu.VMEM((EPW,), jnp.int32),
            pltpu.VMEM((NPAD,), jnp.int32),
            pltpu.SemaphoreType.DMA,
        ],
    )(dst, src)

    gathered = pl.kernel(
        _merge_gather_body,
        out_type=jax.ShapeDtypeStruct((NPAD, D), jnp.float32),
        mesh=mesh,
        compiler_params=sc_params,
        scratch_types=[
            pltpu.VMEM((NW * CPW0,), jnp.int32),
            pltpu.VMEM((NG0, GCHUNK), jnp.int32),
            pltpu.VMEM((CPW0, D), jnp.float32),
            pltpu.SemaphoreType.DMA,
        ],
    )(tables, x_pad)

    return pl.pallas_call(
        _mlp_body,
        grid=(N // BLK,),
        in_specs=[
            pl.BlockSpec((BLK, D), lambda i: (i, 0)),
            pl.BlockSpec((BLK, D), lambda i: (i, 0)),
            pl.BlockSpec((D, D), lambda i: (0, 0)),
            pl.BlockSpec((1, D), lambda i: (0, 0)),
            pl.BlockSpec((D, D), lambda i: (0, 0)),
            pl.BlockSpec((1, D), lambda i: (0, 0)),
        ],
        out_specs=pl.BlockSpec((BLK, D), lambda i: (i, 0)),
        out_shape=jax.ShapeDtypeStruct((N, D), jnp.float32),
    )(gathered, node_features, W1, b1.reshape(1, D), W2, b2.reshape(1, D))
